# baseline (speedup 1.0000x reference)
import jax
import jax.numpy as jnp
from jax import lax
from jax.experimental import pallas as pl
from jax.experimental.pallas import tpu as pltpu
from jax.experimental.pallas import tpu_sc as plsc

B = 16384
C = 1000
K = 200
NC = 2
NS = 16
NW = NC * NS
K_SP = 64
K_ST = K - K_SP
N_SP = K_SP // NC
NBUF = 8
ST_BASE = K_ST // NW
ST_REM = K_ST % NW
MAX_ST = ST_BASE + 1

_vmesh = plsc.VectorSubcoreMesh(
    core_axis_name="c", subcore_axis_name="s", num_cores=NC, num_subcores=NS
)
_smesh = plsc.ScalarSubcoreMesh(axis_name="c", num_cores=NC)

_VMEM_V = pltpu.MemorySpace.VMEM @ _vmesh
_SMEM_S = pltpu.MemorySpace.SMEM @ _smesh
_SEM_V = pltpu.SemaphoreType.DMA @ _vmesh
_SEM_S = pltpu.SemaphoreType.DMA @ _smesh

_scratch = [
    _VMEM_V((K + 24,), jnp.int32),
    *[_VMEM_V((B,), jnp.float32) for _ in range(MAX_ST)],
    *[_SEM_V for _ in range(MAX_ST)],
    _SEM_V,
    _SMEM_S((K,), jnp.int32),
    pltpu.MemorySpace.VMEM_SHARED((NBUF, B), jnp.float32),
    *[_SEM_S for _ in range(NBUF)],
    *[_SEM_S for _ in range(NBUF)],
]


def _tec_body(xt_hbm, mask_hbm, out_hbm, *rest):
    mask_v = rest[0]
    rows = rest[1:1 + MAX_ST]
    sem_in = rest[1 + MAX_ST:1 + 2 * MAX_ST]
    sem_out = rest[1 + 2 * MAX_ST]
    wid = lax.axis_index("s") * NC + lax.axis_index("c")

    pltpu.sync_copy(mask_hbm, mask_v.at[pl.ds(0, K)])
    lane0 = lax.iota(jnp.int32, 16) == 0

    cnt = jnp.where(wid < ST_REM, ST_BASE + 1, ST_BASE)
    start = wid * ST_BASE + jnp.minimum(wid, ST_REM)

    def g_desc(j, i):
        mv = mask_v[pl.ds(j, 16)]
        jsrc = jnp.sum(jnp.where(lane0, mv, 0))
        return pltpu.make_async_copy(xt_hbm.at[jsrc], rows[i], sem_in[i])

    def p_desc(j, i):
        return pltpu.make_async_copy(rows[i], out_hbm.at[j], sem_out)

    for i in range(MAX_ST):
        @pl.when(i < cnt)
        def _():
            g_desc(start + i, i).start()

    for i in range(MAX_ST):
        @pl.when(i < cnt)
        def _():
            g_desc(start + i, i).wait()
            p_desc(start + i, i).start()

    for i in range(MAX_ST):
        @pl.when(i < cnt)
        def _():
            p_desc(start + i, i).wait()


def _scs_body(xt_hbm, mask_hbm, out_hbm, *rest):
    mask_sm = rest[2 + 2 * MAX_ST]
    spmem = rest[3 + 2 * MAX_ST]
    sem_in = rest[4 + 2 * MAX_ST:4 + 2 * MAX_ST + NBUF]
    sem_out = rest[4 + 2 * MAX_ST + NBUF:4 + 2 * MAX_ST + 2 * NBUF]
    cid = lax.axis_index("c")
    start = K_ST + cid * N_SP

    pltpu.sync_copy(mask_hbm, mask_sm)

    def g_desc(i, s):
        return pltpu.make_async_copy(
            xt_hbm.at[mask_sm[start + i]], spmem.at[s], sem_in[s]
        )

    def p_desc(i, s):
        return pltpu.make_async_copy(
            spmem.at[s], out_hbm.at[start + i], sem_out[s]
        )

    for i in range(NBUF):
        g_desc(i, i).start()
    for i in range(N_SP):
        s = i % NBUF
        g_desc(i, s).wait()
        p_desc(i, s).start()
        nxt = i + NBUF
        if nxt < N_SP:
            p_desc(i, s).wait()
            g_desc(nxt, s).start()
    for i in range(N_SP - NBUF, N_SP):
        p_desc(i, i % NBUF).wait()


_row_gather = pl.kernel(
    [_tec_body, _scs_body],
    out_type=jax.ShapeDtypeStruct((K, B), jnp.float32),
    mesh=[_vmesh, _smesh],
    scratch_types=_scratch,
    compiler_params=pltpu.CompilerParams(needs_layout_passes=False),
)


def kernel(x, mask):
    return _row_gather(x.T, mask).T


# speedup vs baseline: 1.2648x; 1.2648x over previous
import functools

import jax
import jax.numpy as jnp
from jax import lax
from jax.experimental import pallas as pl
from jax.experimental.pallas import tpu as pltpu
from jax.experimental.pallas import tpu_sc as plsc

B = 16384
C = 1000
K = 200
NC = 2
NS = 16
NW = NC * NS
BASE_CNT = K // NW
REM = K % NW
MAX_CNT = BASE_CNT + 1

_mesh = plsc.VectorSubcoreMesh(
    core_axis_name="c", subcore_axis_name="s", num_cores=NC, num_subcores=NS
)


@functools.partial(
    pl.kernel,
    out_type=jax.ShapeDtypeStruct((K, B), jnp.float32),
    mesh=_mesh,
    scratch_types=[
        pltpu.VMEM((K + 24,), jnp.int32),
        *[pltpu.VMEM((B,), jnp.float32) for _ in range(MAX_CNT)],
        *[pltpu.SemaphoreType.DMA for _ in range(MAX_CNT)],
        pltpu.SemaphoreType.DMA,
    ],
    compiler_params=pltpu.CompilerParams(needs_layout_passes=False),
)
def _row_gather(xt_hbm, mask_hbm, out_hbm, mask_v, *bufs_and_sems):
    rows = bufs_and_sems[:MAX_CNT]
    sem_in = bufs_and_sems[MAX_CNT:2 * MAX_CNT]
    sem_out = bufs_and_sems[2 * MAX_CNT]
    wid = lax.axis_index("s") * NC + lax.axis_index("c")

    pltpu.sync_copy(mask_hbm, mask_v.at[pl.ds(0, K)])
    lane0 = lax.iota(jnp.int32, 16) == 0

    cnt = jnp.where(wid < REM, BASE_CNT + 1, BASE_CNT)
    start = wid * BASE_CNT + jnp.minimum(wid, REM)

    def g_desc(j, i):
        mv = mask_v[pl.ds(j, 16)]
        jsrc = jnp.sum(jnp.where(lane0, mv, 0))
        return pltpu.make_async_copy(xt_hbm.at[jsrc], rows[i], sem_in[i])

    def p_desc(j, i):
        return pltpu.make_async_copy(rows[i], out_hbm.at[j], sem_out)

    for i in range(MAX_CNT):
        @pl.when(i < cnt)
        def _():
            g_desc(start + i, i).start()

    for i in range(MAX_CNT):
        @pl.when(i < cnt)
        def _():
            g_desc(start + i, i).wait()
            p_desc(start + i, i).start()

    for i in range(MAX_CNT):
        @pl.when(i < cnt)
        def _():
            p_desc(start + i, i).wait()


def kernel(x, mask):
    return _row_gather(x.T, mask).T


# speedup vs baseline: 1.2747x; 1.0078x over previous
import functools

import jax
import jax.numpy as jnp
from jax import lax
from jax.experimental import pallas as pl
from jax.experimental.pallas import tpu as pltpu
from jax.experimental.pallas import tpu_sc as plsc

B = 16384
C = 1000
K = 200
NC = 2
NS = 16
NW = NC * NS
FULL = K // NW
K_FULL = FULL * NW
HALF = B // 2
N_HALF = (K - K_FULL) * 2

_mesh = plsc.VectorSubcoreMesh(
    core_axis_name="c", subcore_axis_name="s", num_cores=NC, num_subcores=NS
)


@functools.partial(
    pl.kernel,
    out_type=jax.ShapeDtypeStruct((K, B), jnp.float32),
    mesh=_mesh,
    scratch_types=[
        pltpu.VMEM((K + 24,), jnp.int32),
        *[pltpu.VMEM((B,), jnp.float32) for _ in range(FULL)],
        pltpu.VMEM((HALF,), jnp.float32),
        *[pltpu.SemaphoreType.DMA for _ in range(FULL + 1)],
        pltpu.SemaphoreType.DMA,
    ],
    compiler_params=pltpu.CompilerParams(needs_layout_passes=False),
)
def _row_gather(xt_hbm, mask_hbm, out_hbm, mask_v, *bufs_and_sems):
    rows = bufs_and_sems[:FULL]
    half_buf = bufs_and_sems[FULL]
    sem_in = bufs_and_sems[FULL + 1:2 * FULL + 2]
    sem_out = bufs_and_sems[2 * FULL + 2]
    wid = lax.axis_index("s") * NC + lax.axis_index("c")

    pltpu.sync_copy(mask_hbm, mask_v.at[pl.ds(0, K)])
    lane0 = lax.iota(jnp.int32, 16) == 0

    def src_row(j):
        mv = mask_v[pl.ds(j, 16)]
        return jnp.sum(jnp.where(lane0, mv, 0))

    start = wid * FULL
    hw = wid - (NW - N_HALF)
    hj = K_FULL + hw // 2
    hoff = (hw % 2) * HALF

    def g_desc(j, i):
        return pltpu.make_async_copy(xt_hbm.at[src_row(j)], rows[i], sem_in[i])

    def p_desc(j, i):
        return pltpu.make_async_copy(rows[i], out_hbm.at[j], sem_out)

    def gh_desc():
        return pltpu.make_async_copy(
            xt_hbm.at[src_row(hj), pl.ds(hoff, HALF)], half_buf, sem_in[FULL]
        )

    def ph_desc():
        return pltpu.make_async_copy(
            half_buf, out_hbm.at[hj, pl.ds(hoff, HALF)], sem_out
        )

    @pl.when(hw >= 0)
    def _():
        gh_desc().start()
    for i in range(FULL):
        g_desc(start + i, i).start()

    for i in range(FULL):
        g_desc(start + i, i).wait()
        p_desc(start + i, i).start()

    @pl.when(hw >= 0)
    def _():
        gh_desc().wait()
        ph_desc().start()

    for i in range(FULL):
        p_desc(start + i, i).wait()

    @pl.when(hw >= 0)
    def _():
        ph_desc().wait()


def kernel(x, mask):
    return _row_gather(x.T, mask).T
